# baseline (speedup 1.0000x reference)
import jax
import jax.numpy as jnp
from jax import lax
from jax.experimental import pallas as pl
from jax.experimental.pallas import tpu as pltpu
from jax.experimental.pallas import tpu_sc as plsc

_INFO = plsc.get_sparse_core_info()
_NC = _INFO.num_cores
_NS = _INFO.num_subcores
_NW = _NC * _NS

_V = 1000000
_VPAD = 1 << 20
_BATCH = 16384
_HIST = 200
_HPAD = 256
_ROWS_PER_W = _BATCH // _NW
_CHUNK_ROWS = 64
_STEPS = _ROWS_PER_W // _CHUNK_ROWS
_TAB_SLICE = _VPAD // _NS


def _gather_body(table_hbm, idx_hbm, out_hbm, tab_s,
                 idx0, idx1, rows0, rows1,
                 sl0, sl1, sg0, sg1, ss0, ss1):
    cid = lax.axis_index("c")
    sid = lax.axis_index("s")
    wid = sid * _NC + cid
    rbase = wid * _ROWS_PER_W

    idx_v = (idx0, idx1)
    rows_v = (rows0, rows1)
    sem_l = (sl0, sl1)
    sem_g = (sg0, sg1)
    sem_s = (ss0, ss1)

    def load_issue(i):
        b = i % 2
        r = rbase + i * _CHUNK_ROWS

        def row(j, carry):
            pltpu.async_copy(idx_hbm.at[r + j],
                             idx_v[b].at[pl.ds(j * _HPAD, _HPAD)], sem_l[b])
            return carry

        lax.fori_loop(0, _CHUNK_ROWS, row, 0)

    def load_drain(i):
        b = i % 2
        r = rbase + i * _CHUNK_ROWS

        def row(j, carry):
            pltpu.make_async_copy(
                idx_hbm.at[r + j],
                idx_v[b].at[pl.ds(j * _HPAD, _HPAD)], sem_l[b]).wait()
            return carry

        lax.fori_loop(0, _CHUNK_ROWS, row, 0)

    def gather(i):
        return pltpu.async_copy(tab_s.at[idx_v[i % 2]],
                                rows_v[i % 2], sem_g[i % 2])

    def store_issue(i):
        b = i % 2
        r = rbase + i * _CHUNK_ROWS

        def row(j, carry):
            pltpu.async_copy(rows_v[b].at[pl.ds(j * _HPAD, _HPAD)],
                             out_hbm.at[r + j], sem_s[b])
            return carry

        lax.fori_loop(0, _CHUNK_ROWS, row, 0)

    def store_drain(i):
        b = i % 2
        r = rbase + i * _CHUNK_ROWS

        def row(j, carry):
            pltpu.make_async_copy(rows_v[b].at[pl.ds(j * _HPAD, _HPAD)],
                                  out_hbm.at[r + j], sem_s[b]).wait()
            return carry

        lax.fori_loop(0, _CHUNK_ROWS, row, 0)

    load_issue(0)

    tb = sid * _TAB_SLICE
    pltpu.sync_copy(table_hbm.at[pl.ds(tb, _TAB_SLICE)],
                    tab_s.at[pl.ds(tb, _TAB_SLICE)])
    plsc.subcore_barrier()

    dma_g = {}
    for i in range(_STEPS):
        load_drain(i)
        if i >= 2:
            store_drain(i - 2)
        dma_g[i] = gather(i)
        if i >= 1:
            dma_g[i - 1].wait()
            store_issue(i - 1)
        if i + 1 < _STEPS:
            load_issue(i + 1)
    dma_g[_STEPS - 1].wait()
    store_issue(_STEPS - 1)
    store_drain(_STEPS - 2)
    store_drain(_STEPS - 1)


@jax.jit
def _sc_gather(table_padded, idx_padded):
    mesh = plsc.VectorSubcoreMesh(core_axis_name="c", subcore_axis_name="s")
    f = pl.kernel(
        _gather_body,
        mesh=mesh,
        out_type=jax.ShapeDtypeStruct((_BATCH, _HPAD), jnp.float32),
        scratch_types=[
            pltpu.VMEM_SHARED((_VPAD,), jnp.float32),
            pltpu.VMEM((_CHUNK_ROWS * _HPAD,), jnp.int32),
            pltpu.VMEM((_CHUNK_ROWS * _HPAD,), jnp.int32),
            pltpu.VMEM((_CHUNK_ROWS * _HPAD,), jnp.float32),
            pltpu.VMEM((_CHUNK_ROWS * _HPAD,), jnp.float32),
            pltpu.SemaphoreType.DMA,
            pltpu.SemaphoreType.DMA,
            pltpu.SemaphoreType.DMA,
            pltpu.SemaphoreType.DMA,
            pltpu.SemaphoreType.DMA,
            pltpu.SemaphoreType.DMA,
        ],
    )
    return f(table_padded, idx_padded)


def kernel(table, user_ids, item_ids):
    table_padded = jnp.pad(table, (0, _VPAD - table.shape[0]))
    idx_padded = jnp.pad(item_ids.astype(jnp.int32),
                         ((0, 0), (0, _HPAD - _HIST)))
    out = _sc_gather(table_padded, idx_padded)
    return out[:, :_HIST]


# speedup vs baseline: 2.4780x; 2.4780x over previous
import jax
import jax.numpy as jnp
from jax import lax
from jax.experimental import pallas as pl
from jax.experimental.pallas import tpu as pltpu
from jax.experimental.pallas import tpu_sc as plsc

_INFO = plsc.get_sparse_core_info()
_NC = _INFO.num_cores
_NS = _INFO.num_subcores
_NW = _NC * _NS

_VPAD = 1 << 20
_B = 16384 * 200
_B_PER_W = _B // _NW
_CHUNK = 12800
_STEPS = _B_PER_W // _CHUNK
_TAB_SLICE = _VPAD // _NS


def _gather_body(table_hbm, idx_hbm, out_hbm, tab_s,
                 idx0, idx1, rows0, rows1,
                 sl0, sl1, sg0, sg1, ss0, ss1):
    cid = lax.axis_index("c")
    sid = lax.axis_index("s")
    wid = sid * _NC + cid
    base = wid * _B_PER_W

    idx_v = (idx0, idx1)
    rows_v = (rows0, rows1)
    sem_l = (sl0, sl1)
    sem_g = (sg0, sg1)
    sem_s = (ss0, ss1)

    def load(i):
        off = base + i * _CHUNK
        return pltpu.async_copy(idx_hbm.at[pl.ds(off, _CHUNK)],
                                idx_v[i % 2], sem_l[i % 2])

    def gather(i):
        return pltpu.async_copy(tab_s.at[idx_v[i % 2]],
                                rows_v[i % 2], sem_g[i % 2])

    def store(i):
        off = base + i * _CHUNK
        return pltpu.async_copy(rows_v[i % 2],
                                out_hbm.at[pl.ds(off, _CHUNK)], sem_s[i % 2])

    dma_l = {0: load(0)}
    dma_g, dma_s = {}, {}

    tb = sid * _TAB_SLICE
    pltpu.sync_copy(table_hbm.at[pl.ds(tb, _TAB_SLICE)],
                    tab_s.at[pl.ds(tb, _TAB_SLICE)])
    plsc.subcore_barrier()

    for i in range(_STEPS):
        dma_l[i].wait()
        if i >= 2:
            dma_s[i - 2].wait()
        dma_g[i] = gather(i)
        if i >= 1:
            dma_g[i - 1].wait()
            dma_s[i - 1] = store(i - 1)
        if i + 1 < _STEPS:
            dma_l[i + 1] = load(i + 1)
    dma_g[_STEPS - 1].wait()
    dma_s[_STEPS - 1] = store(_STEPS - 1)
    dma_s[_STEPS - 2].wait()
    dma_s[_STEPS - 1].wait()


@jax.jit
def _sc_gather(table_padded, idx_flat):
    mesh = plsc.VectorSubcoreMesh(core_axis_name="c", subcore_axis_name="s")
    f = pl.kernel(
        _gather_body,
        mesh=mesh,
        out_type=jax.ShapeDtypeStruct((_B,), jnp.float32),
        scratch_types=[
            pltpu.VMEM_SHARED((_VPAD,), jnp.float32),
            pltpu.VMEM((_CHUNK,), jnp.int32),
            pltpu.VMEM((_CHUNK,), jnp.int32),
            pltpu.VMEM((_CHUNK,), jnp.float32),
            pltpu.VMEM((_CHUNK,), jnp.float32),
            pltpu.SemaphoreType.DMA,
            pltpu.SemaphoreType.DMA,
            pltpu.SemaphoreType.DMA,
            pltpu.SemaphoreType.DMA,
            pltpu.SemaphoreType.DMA,
            pltpu.SemaphoreType.DMA,
        ],
    )
    return f(table_padded, idx_flat)


def kernel(table, user_ids, item_ids):
    table_padded = jnp.pad(table, (0, _VPAD - table.shape[0]))
    idx_flat = item_ids.reshape(-1).astype(jnp.int32)
    out = _sc_gather(table_padded, idx_flat)
    return out.reshape(item_ids.shape)
